# baseline (speedup 1.0000x reference)
import functools

import jax
import jax.numpy as jnp
from jax import lax
from jax.experimental import pallas as pl
from jax.experimental.pallas import tpu as pltpu
from jax.experimental.pallas import tpu_sc as plsc

NUM_HEADS = 16
NUM_BUCKETS = 32
MAX_DISTANCE = 128
B = 2
N = 1024

LUT_A = 256
LUT_SIZE = LUT_A * LUT_A
IDX_OFFSET = 127 * 256 + 127

LANES = 16
NUM_PAIRS = NUM_HEADS // 2
ROW_HALF = N // 2
ROWS_PER_SLAB = 8
NUM_SLABS = ROW_HALF // ROWS_PER_SLAB
CHUNKS = N // LANES


def _lut_body(bias_ref, out_ref):
    nb = NUM_BUCKETS // 2
    max_exact = nb // 2

    def bucket(rel):
        n = -rel
        ret = (n < 0).astype(jnp.int32) * nb
        n = jnp.abs(n)
        is_small = n < max_exact
        safe_n = jnp.maximum(n, 1).astype(jnp.float32)
        val_if_large = max_exact + (
            jnp.log(safe_n / max_exact)
            / jnp.log(jnp.float32(MAX_DISTANCE / max_exact))
            * (nb - max_exact)
        ).astype(jnp.int32)
        val_if_large = jnp.minimum(val_if_large, nb - 1)
        return ret + jnp.where(is_small, n, val_if_large)

    f_col = bucket(lax.broadcasted_iota(jnp.int32, (LUT_A, 1), 0) - 127)
    oh_a = (f_col == lax.broadcasted_iota(jnp.int32, (LUT_A, NUM_BUCKETS), 1))
    oh_a = oh_a.astype(jnp.float32)
    half = LUT_A // 2
    oh_bt_u = []
    for u in range(2):
        f_row = bucket(
            lax.broadcasted_iota(jnp.int32, (1, half), 1) + (u * half - 127))
        oh = (lax.broadcasted_iota(jnp.int32, (NUM_BUCKETS, half), 0) == f_row)
        oh_bt_u.append(oh.astype(jnp.float32))

    p2 = jnp.dot(oh_a, bias_ref[...], preferred_element_type=jnp.float32)

    def head_lut(h, u):
        p = p2[:, h * NUM_BUCKETS:(h + 1) * NUM_BUCKETS]
        return jnp.dot(p, oh_bt_u[u], preferred_element_type=jnp.float32)

    for u in range(2):
        for hp in range(NUM_PAIRS):
            lo = lax.bitcast_convert_type(
                head_lut(2 * hp, u).astype(jnp.bfloat16), jnp.uint16
            ).astype(jnp.int32)
            hi = lax.bitcast_convert_type(
                head_lut(2 * hp + 1, u).astype(jnp.bfloat16), jnp.uint16
            ).astype(jnp.int32)
            out_ref[u, :, hp, :] = lo | (hi << 16)


_build_lut = pl.pallas_call(
    _lut_body,
    out_shape=jax.ShapeDtypeStruct((2, LUT_A, NUM_PAIRS, LUT_A // 2), jnp.int32),
)


@functools.cache
def _make_sc_gather():
    mesh = plsc.VectorSubcoreMesh(core_axis_name="c", subcore_axis_name="s")
    return functools.partial(
        pl.kernel,
        mesh=mesh,
        out_type=jax.ShapeDtypeStruct((B * NUM_HEADS * N, N), jnp.float32),
        scratch_types=[
            pltpu.VMEM((2 * N,), jnp.float32),
            pltpu.VMEM((N,), jnp.int32),
            pltpu.VMEM((N,), jnp.int32),
            pltpu.VMEM((2, LUT_A, LUT_A // 2), jnp.int32),
            pltpu.VMEM((2, ROWS_PER_SLAB, N), jnp.float32),
            pltpu.VMEM((2, ROWS_PER_SLAB, N), jnp.float32),
            pltpu.SemaphoreType.DMA,
            pltpu.SemaphoreType.DMA,
        ],
        compiler_params=pltpu.CompilerParams(needs_layout_passes=False),
    )(_sc_gather_body)


def _sc_gather_body(lut_hbm, coords_hbm, out_hbm,
                    coords_v, p_v, q_v, lut_v, bufs0, bufs1, sem0, sem1):
    b = lax.axis_index("c")
    s_idx = lax.axis_index("s")
    hp = s_idx % NUM_PAIRS
    rh = s_idx // NUM_PAIRS

    pltpu.sync_copy(coords_hbm.at[b], coords_v)
    pltpu.sync_copy(lut_hbm.at[:, :, hp, :], lut_v)

    lanes = lax.broadcasted_iota(jnp.int32, (LANES,), 0)

    @plsc.parallel_loop(0, CHUNKS, 1, unroll=2)
    def _init_body(c):
        base = c * LANES
        xi = plsc.load_gather(coords_v, [(lanes + base) * 2])
        yi = plsc.load_gather(coords_v, [(lanes + base) * 2 + 1])
        px = (xi * jnp.float32(MAX_DISTANCE)).astype(jnp.int32)
        py = (yi * jnp.float32(MAX_DISTANCE)).astype(jnp.int32)
        pv = px * 256 + py
        p_v[pl.ds(base, LANES)] = pv
        q_v[pl.ds(base, LANES)] = IDX_OFFSET - pv

    sems = (sem0, sem1)
    row_first = rh * ROW_HALF
    out_row0 = (b * NUM_HEADS + 2 * hp) * N + row_first
    out_row1 = (b * NUM_HEADS + 2 * hp + 1) * N + row_first

    def do_slab(s, buf0, buf1):
        base_row = row_first + s * ROWS_PER_SLAB
        splats = [
            plsc.load_gather(p_v, [jnp.full((LANES,), base_row + r, jnp.int32)])
            for r in range(ROWS_PER_SLAB)
        ]

        @plsc.parallel_loop(0, CHUNKS, 1, unroll=2)
        def _col_body(c):
            q = q_v[pl.ds(c * LANES, LANES)]
            for r in range(ROWS_PER_SLAB):
                idx = splats[r] + q
                g = plsc.load_gather(
                    lut_v, [(idx >> 7) & 1, idx >> 8, idx & 127])
                gb = plsc.bitcast(g, jnp.bfloat16)
                v0, v1 = plsc.unpack(gb, format=plsc.PackFormat.INTERLEAVED)
                buf0[r, pl.ds(c * LANES, LANES)] = v0
                buf1[r, pl.ds(c * LANES, LANES)] = v1

    def pair_body(ss, carry):
        for k in range(2):
            s = ss * 2 + k
            off = s * ROWS_PER_SLAB
            dst0 = out_hbm.at[pl.ds(out_row0 + off, ROWS_PER_SLAB), :]
            dst1 = out_hbm.at[pl.ds(out_row1 + off, ROWS_PER_SLAB), :]

            @pl.when(ss > 0)
            def _wait():
                pltpu.make_async_copy(bufs0.at[k], dst0, sems[k]).wait()
                pltpu.make_async_copy(bufs1.at[k], dst1, sems[k]).wait()

            do_slab(s, bufs0.at[k], bufs1.at[k])
            pltpu.async_copy(bufs0.at[k], dst0, sems[k])
            pltpu.async_copy(bufs1.at[k], dst1, sems[k])
        return carry

    lax.fori_loop(0, NUM_SLABS // 2, pair_body, 0)

    tail0 = out_hbm.at[pl.ds(out_row0, ROWS_PER_SLAB), :]
    tail1 = out_hbm.at[pl.ds(out_row1, ROWS_PER_SLAB), :]
    for k in range(2):
        pltpu.make_async_copy(bufs0.at[k], tail0, sems[k]).wait()
        pltpu.make_async_copy(bufs1.at[k], tail1, sems[k]).wait()


def kernel(coords_2d, bias_table):
    bias_r = bias_table.reshape(
        NUM_BUCKETS, NUM_BUCKETS, NUM_HEADS
    ).transpose(0, 2, 1).reshape(NUM_BUCKETS, NUM_HEADS * NUM_BUCKETS)
    lut = _build_lut(bias_r)
    coords_flat = coords_2d.reshape(B, 2 * N)
    out = _make_sc_gather()(lut, coords_flat)
    return out.reshape(B, NUM_HEADS, N, N)


# speedup vs baseline: 1.3203x; 1.3203x over previous
import functools

import jax
import jax.numpy as jnp
from jax import lax
from jax.experimental import pallas as pl
from jax.experimental.pallas import tpu as pltpu
from jax.experimental.pallas import tpu_sc as plsc

NUM_HEADS = 16
NUM_BUCKETS = 32
MAX_DISTANCE = 128
B = 2
N = 1024

LUT_A = 256
LUT_SIZE = LUT_A * LUT_A
IDX_OFFSET = 127 * 256 + 127

LANES = 16
NUM_PAIRS = NUM_HEADS // 2
ROW_HALF = N // 2
ROWS_PER_SLAB = 8
NUM_SLABS = ROW_HALF // ROWS_PER_SLAB
CHUNKS = N // LANES
PB_STRIDE = NUM_BUCKETS * NUM_BUCKETS + 1


def _bucket_body(out_ref):
    nb = NUM_BUCKETS // 2
    max_exact = nb // 2
    rel = lax.broadcasted_iota(jnp.int32, (1, LUT_A), 1) - 127
    n = -rel
    ret = (n < 0).astype(jnp.int32) * nb
    n = jnp.abs(n)
    is_small = n < max_exact
    safe_n = jnp.maximum(n, 1).astype(jnp.float32)
    val_if_large = max_exact + (
        jnp.log(safe_n / max_exact)
        / jnp.log(jnp.float32(MAX_DISTANCE / max_exact))
        * (nb - max_exact)
    ).astype(jnp.int32)
    val_if_large = jnp.minimum(val_if_large, nb - 1)
    out_ref[...] = ret + jnp.where(is_small, n, val_if_large)


_build_bucket = pl.pallas_call(
    _bucket_body,
    out_shape=jax.ShapeDtypeStruct((1, LUT_A), jnp.int32),
)


@functools.cache
def _make_sc_gather():
    mesh = plsc.VectorSubcoreMesh(core_axis_name="c", subcore_axis_name="s")
    return functools.partial(
        pl.kernel,
        mesh=mesh,
        out_type=jax.ShapeDtypeStruct((B * NUM_HEADS * N, N), jnp.float32),
        scratch_types=[
            pltpu.VMEM((2 * N,), jnp.float32),
            pltpu.VMEM((N,), jnp.int32),
            pltpu.VMEM((N,), jnp.int32),
            pltpu.VMEM((LUT_A,), jnp.int32),
            pltpu.VMEM((N,), jnp.float32),
            pltpu.VMEM((N,), jnp.float32),
            pltpu.VMEM((16 * PB_STRIDE,), jnp.int32),
            pltpu.VMEM((LUT_SIZE,), jnp.int32),
            pltpu.VMEM((2, ROWS_PER_SLAB, N), jnp.float32),
            pltpu.VMEM((2, ROWS_PER_SLAB, N), jnp.float32),
            pltpu.SemaphoreType.DMA,
            pltpu.SemaphoreType.DMA,
        ],
        compiler_params=pltpu.CompilerParams(needs_layout_passes=False),
    )(_sc_gather_body)


def _sc_gather_body(f_hbm, bias_t_hbm, coords_hbm, out_hbm,
                    coords_v, p_v, q_v, f_v, col0_v, col1_v, pb_v, lut_v,
                    bufs0, bufs1, sem0, sem1):
    b = lax.axis_index("c")
    s_idx = lax.axis_index("s")
    hp = s_idx % NUM_PAIRS
    rh = s_idx // NUM_PAIRS

    pltpu.sync_copy(coords_hbm.at[b], coords_v)
    pltpu.sync_copy(f_hbm.at[0], f_v)
    pltpu.sync_copy(bias_t_hbm.at[2 * hp], col0_v)
    pltpu.sync_copy(bias_t_hbm.at[2 * hp + 1], col1_v)

    lanes = lax.broadcasted_iota(jnp.int32, (LANES,), 0)

    @plsc.parallel_loop(0, NUM_BUCKETS * NUM_BUCKETS // LANES, 1, unroll=2)
    def _pack_body(k):
        base = k * LANES
        a = col0_v[pl.ds(base, LANES)]
        bb = col1_v[pl.ds(base, LANES)]
        packed = plsc.bitcast(
            plsc.pack(a, bb, format=plsc.PackFormat.INTERLEAVED), jnp.int32)
        for rep in range(16):
            pb_v[pl.ds(rep * PB_STRIDE + base, LANES)] = packed

    @plsc.parallel_loop(0, CHUNKS, 1, unroll=2)
    def _init_body(c):
        base = c * LANES
        xi = plsc.load_gather(coords_v, [(lanes + base) * 2])
        yi = plsc.load_gather(coords_v, [(lanes + base) * 2 + 1])
        px = (xi * jnp.float32(MAX_DISTANCE)).astype(jnp.int32)
        py = (yi * jnp.float32(MAX_DISTANCE)).astype(jnp.int32)
        pv = px * 256 + py
        p_v[pl.ds(base, LANES)] = pv
        q_v[pl.ds(base, LANES)] = IDX_OFFSET - pv

    lane_off = lanes * PB_STRIDE

    @plsc.parallel_loop(0, LUT_A, 1, unroll=2)
    def _build_body(a):
        fa32 = plsc.load_gather(
            f_v, [jnp.full((LANES,), a, jnp.int32)]) * NUM_BUCKETS + lane_off
        for c0 in range(LUT_A // LANES):
            fc = f_v[pl.ds(c0 * LANES, LANES)]
            val = plsc.load_gather(pb_v, [fa32 + fc])
            lut_v[pl.ds(a * LUT_A + c0 * LANES, LANES)] = val

    sems = (sem0, sem1)
    row_first = rh * ROW_HALF
    out_row0 = (b * NUM_HEADS + 2 * hp) * N + row_first
    out_row1 = (b * NUM_HEADS + 2 * hp + 1) * N + row_first

    def do_slab(s, buf0, buf1):
        base_row = row_first + s * ROWS_PER_SLAB
        splats = [
            plsc.load_gather(p_v, [jnp.full((LANES,), base_row + r, jnp.int32)])
            for r in range(ROWS_PER_SLAB)
        ]

        @plsc.parallel_loop(0, CHUNKS, 1, unroll=2)
        def _col_body(c):
            q = q_v[pl.ds(c * LANES, LANES)]
            for r in range(ROWS_PER_SLAB):
                idx = splats[r] + q
                g = plsc.load_gather(lut_v, [idx])
                gb = plsc.bitcast(g, jnp.bfloat16)
                v0, v1 = plsc.unpack(gb, format=plsc.PackFormat.INTERLEAVED)
                buf0[r, pl.ds(c * LANES, LANES)] = v0
                buf1[r, pl.ds(c * LANES, LANES)] = v1

    def pair_body(ss, carry):
        for k in range(2):
            s = ss * 2 + k
            off = s * ROWS_PER_SLAB
            dst0 = out_hbm.at[pl.ds(out_row0 + off, ROWS_PER_SLAB), :]
            dst1 = out_hbm.at[pl.ds(out_row1 + off, ROWS_PER_SLAB), :]

            @pl.when(ss > 0)
            def _wait():
                pltpu.make_async_copy(bufs0.at[k], dst0, sems[k]).wait()
                pltpu.make_async_copy(bufs1.at[k], dst1, sems[k]).wait()

            do_slab(s, bufs0.at[k], bufs1.at[k])
            pltpu.async_copy(bufs0.at[k], dst0, sems[k])
            pltpu.async_copy(bufs1.at[k], dst1, sems[k])
        return carry

    lax.fori_loop(0, NUM_SLABS // 2, pair_body, 0)

    tail0 = out_hbm.at[pl.ds(out_row0, ROWS_PER_SLAB), :]
    tail1 = out_hbm.at[pl.ds(out_row1, ROWS_PER_SLAB), :]
    for k in range(2):
        pltpu.make_async_copy(bufs0.at[k], tail0, sems[k]).wait()
        pltpu.make_async_copy(bufs1.at[k], tail1, sems[k]).wait()


def kernel(coords_2d, bias_table):
    f = _build_bucket()
    bias_t = bias_table.T
    coords_flat = coords_2d.reshape(B, 2 * N)
    out = _make_sc_gather()(f, bias_t, coords_flat)
    return out.reshape(B, NUM_HEADS, N, N)
